# Initial kernel scaffold; baseline (speedup 1.0000x reference)
import jax
import jax.numpy as jnp
from jax.experimental import pallas as pl


def kernel(x, coords, Wx, Wy):
    raise NotImplementedError("write your pallas kernel here")


# speedup vs baseline: 1.1145x; 1.1145x over previous
import functools

import jax
import jax.numpy as jnp
from jax import lax
from jax.experimental import pallas as pl
from jax.experimental.pallas import tpu as pltpu
from jax.experimental.pallas import tpu_sc as plsc

SEQ = 16384
DIM = 64
HALF = 32
MAX_LEN = 512
NC = 2
NS = 16
L = 16
CHUNK = SEQ // NS
ROWS = SEQ // (NC * NS)
INT_MAX = 2147483647

_mesh = plsc.VectorSubcoreMesh(core_axis_name="c", subcore_axis_name="s")


@functools.partial(
    pl.kernel,
    out_type=jax.ShapeDtypeStruct((SEQ * DIM,), jnp.float32),
    mesh=_mesh,
    compiler_params=pltpu.CompilerParams(needs_layout_passes=False),
    scratch_types=[
        pltpu.VMEM((CHUNK,), jnp.int32),
        pltpu.VMEM((CHUNK,), jnp.int32),
        pltpu.VMEM((L,), jnp.int32),
        pltpu.VMEM((L,), jnp.int32),
        pltpu.VMEM_SHARED((NS * L,), jnp.int32),
        pltpu.VMEM_SHARED((NS * L,), jnp.int32),
        pltpu.VMEM((NS * L,), jnp.int32),
        pltpu.VMEM((NS * L,), jnp.int32),
        pltpu.VMEM((2 * L,), jnp.int32),
        pltpu.VMEM((2 * L,), jnp.int32),
        pltpu.VMEM((MAX_LEN * HALF,), jnp.float32),
        pltpu.VMEM((MAX_LEN * HALF,), jnp.float32),
        pltpu.VMEM((ROWS * DIM,), jnp.float32),
        pltpu.SemaphoreType.DMA,
    ],
)
def _pe_kernel(cx, cy, x_flat, wx, wy, out, cx_v, cy_v, stx, sty,
               minx_sh, miny_sh, mgx, mgy, redx, redy,
               wx_v, wy_v, xo_v, sem):
    c = lax.axis_index("c")
    s = lax.axis_index("s")
    base = (s * CHUNK + c * ROWS) * DIM
    cp_x = pltpu.async_copy(x_flat.at[pl.ds(base, ROWS * DIM)], xo_v, sem)
    cp_wx = pltpu.async_copy(wx, wx_v, sem)
    cp_wy = pltpu.async_copy(wy, wy_v, sem)

    pltpu.sync_copy(cx.at[pl.ds(s * CHUNK, CHUNK)], cx_v)
    pltpu.sync_copy(cy.at[pl.ds(s * CHUNK, CHUNK)], cy_v)

    def min_body(j, carry):
        mx, my = carry
        vx = cx_v[pl.ds(j * L, L)]
        vy = cy_v[pl.ds(j * L, L)]
        return jnp.minimum(mx, vx), jnp.minimum(my, vy)

    init = (jnp.full((L,), INT_MAX, jnp.int32), jnp.full((L,), INT_MAX, jnp.int32))
    mx, my = lax.fori_loop(0, CHUNK // L, min_body, init)

    stx[...] = mx
    sty[...] = my
    pltpu.sync_copy(stx, minx_sh.at[pl.ds(s * L, L)])
    pltpu.sync_copy(sty, miny_sh.at[pl.ds(s * L, L)])
    plsc.subcore_barrier()
    pltpu.sync_copy(minx_sh, mgx)
    pltpu.sync_copy(miny_sh, mgy)
    vx = mgx[pl.ds(0, L)]
    vy = mgy[pl.ds(0, L)]
    for j in range(1, NS):
        vx = jnp.minimum(vx, mgx[pl.ds(j * L, L)])
        vy = jnp.minimum(vy, mgy[pl.ds(j * L, L)])
    redx[pl.ds(0, L)] = vx
    redx[pl.ds(L, L)] = vx
    redy[pl.ds(0, L)] = vy
    redy[pl.ds(L, L)] = vy
    gmx = redx[pl.ds(0, L)]
    gmy = redy[pl.ds(0, L)]
    for k in range(1, L):
        gmx = jnp.minimum(gmx, redx[pl.ds(k, L)])
        gmy = jnp.minimum(gmy, redy[pl.ds(k, L)])

    cp_x.wait()
    cp_wx.wait()
    cp_wy.wait()

    off = c * ROWS
    shift4 = jnp.full((L,), 4, jnp.int32)
    shift5 = jnp.full((L,), 5, jnp.int32)
    lanes = lax.iota(jnp.int32, L)

    def add_body(j, _):
        vx = cx_v[pl.ds(off + j * L, L)]
        vy = cy_v[pl.ds(off + j * L, L)]
        gxv = lax.shift_left(lax.shift_right_logical(vx - gmx, shift4), shift5)
        gyv = lax.shift_left(lax.shift_right_logical(vy - gmy, shift4), shift5)
        rowvec = (j * L + lanes) * DIM
        for k in range(HALF):
            xi = rowvec + k
            xv = plsc.load_gather(xo_v, [xi])
            ex = plsc.load_gather(wx_v, [gxv + k])
            plsc.store_scatter(xo_v, [xi], xv + ex)
        for k in range(HALF):
            xi = rowvec + HALF + k
            xv = plsc.load_gather(xo_v, [xi])
            ey = plsc.load_gather(wy_v, [gyv + k])
            plsc.store_scatter(xo_v, [xi], xv + ey)
        return 0

    lax.fori_loop(0, ROWS // L, add_body, 0)
    pltpu.sync_copy(xo_v, out.at[pl.ds(base, ROWS * DIM)])


def kernel(x, coords, Wx, Wy):
    out = _pe_kernel(coords[:, 1], coords[:, 2], x.reshape(-1),
                     Wx.reshape(-1), Wy.reshape(-1))
    return out.reshape(SEQ, DIM)


# speedup vs baseline: 1.3732x; 1.2321x over previous
import functools

import jax
import jax.numpy as jnp
from jax import lax
from jax.experimental import pallas as pl
from jax.experimental.pallas import tpu as pltpu
from jax.experimental.pallas import tpu_sc as plsc

SEQ = 16384
DIM = 64
HALF = 32
MAX_LEN = 512
NC = 2
NS = 16
L = 16
CHUNK = SEQ // NS
ROWS = SEQ // (NC * NS)
INT_MAX = 2147483647

_mesh = plsc.VectorSubcoreMesh(core_axis_name="c", subcore_axis_name="s")


@functools.partial(
    pl.kernel,
    out_type=jax.ShapeDtypeStruct((SEQ * DIM,), jnp.float32),
    mesh=_mesh,
    compiler_params=pltpu.CompilerParams(needs_layout_passes=False),
    scratch_types=[
        pltpu.VMEM((CHUNK,), jnp.int32),
        pltpu.VMEM((CHUNK,), jnp.int32),
        pltpu.VMEM((L,), jnp.int32),
        pltpu.VMEM((L,), jnp.int32),
        pltpu.VMEM_SHARED((NS * L,), jnp.int32),
        pltpu.VMEM_SHARED((NS * L,), jnp.int32),
        pltpu.VMEM((NS * L,), jnp.int32),
        pltpu.VMEM((NS * L,), jnp.int32),
        pltpu.VMEM((2 * L,), jnp.int32),
        pltpu.VMEM((2 * L,), jnp.int32),
        pltpu.VMEM((MAX_LEN * HALF,), jnp.float32),
        pltpu.VMEM((MAX_LEN * HALF,), jnp.float32),
        pltpu.VMEM((ROWS * DIM,), jnp.float32),
        pltpu.SemaphoreType.DMA,
    ],
)
def _pe_kernel(cx, cy, x_flat, wx, wy, out, cx_v, cy_v, stx, sty,
               minx_sh, miny_sh, mgx, mgy, redx, redy,
               wx_v, wy_v, xo_v, sem):
    c = lax.axis_index("c")
    s = lax.axis_index("s")
    base = (s * CHUNK + c * ROWS) * DIM
    cp_x = pltpu.async_copy(x_flat.at[pl.ds(base, ROWS * DIM)], xo_v, sem)
    cp_wx = pltpu.async_copy(wx, wx_v, sem)
    cp_wy = pltpu.async_copy(wy, wy_v, sem)

    pltpu.sync_copy(cx.at[pl.ds(s * CHUNK, CHUNK)], cx_v)
    pltpu.sync_copy(cy.at[pl.ds(s * CHUNK, CHUNK)], cy_v)

    def min_body(j, carry):
        mx, my = carry
        vx = cx_v[pl.ds(j * L, L)]
        vy = cy_v[pl.ds(j * L, L)]
        return jnp.minimum(mx, vx), jnp.minimum(my, vy)

    init = (jnp.full((L,), INT_MAX, jnp.int32), jnp.full((L,), INT_MAX, jnp.int32))
    mx, my = lax.fori_loop(0, CHUNK // L, min_body, init)

    stx[...] = mx
    sty[...] = my
    pltpu.sync_copy(stx, minx_sh.at[pl.ds(s * L, L)])
    pltpu.sync_copy(sty, miny_sh.at[pl.ds(s * L, L)])
    plsc.subcore_barrier()
    pltpu.sync_copy(minx_sh, mgx)
    pltpu.sync_copy(miny_sh, mgy)
    vx = mgx[pl.ds(0, L)]
    vy = mgy[pl.ds(0, L)]
    for j in range(1, NS):
        vx = jnp.minimum(vx, mgx[pl.ds(j * L, L)])
        vy = jnp.minimum(vy, mgy[pl.ds(j * L, L)])
    redx[pl.ds(0, L)] = vx
    redx[pl.ds(L, L)] = vx
    redy[pl.ds(0, L)] = vy
    redy[pl.ds(L, L)] = vy
    gmx = redx[pl.ds(0, L)]
    gmy = redy[pl.ds(0, L)]
    for k in range(1, L):
        gmx = jnp.minimum(gmx, redx[pl.ds(k, L)])
        gmy = jnp.minimum(gmy, redy[pl.ds(k, L)])

    cp_x.wait()
    cp_wx.wait()
    cp_wy.wait()

    off = c * ROWS
    shift4 = jnp.full((L,), 4, jnp.int32)
    shift5 = jnp.full((L,), 5, jnp.int32)
    lanes = lax.iota(jnp.int32, L)

    @plsc.parallel_loop(0, ROWS // L, 1, unroll=2)
    def add_body(j):
        vx = cx_v[pl.ds(off + j * L, L)]
        vy = cy_v[pl.ds(off + j * L, L)]
        gxv = lax.shift_left(lax.shift_right_logical(vx - gmx, shift4), shift5)
        gyv = lax.shift_left(lax.shift_right_logical(vy - gmy, shift4), shift5)
        rowvec = (j * L + lanes) * DIM
        rowvec2 = rowvec + HALF
        for kb in range(0, HALF, 4):
            xs = [plsc.load_gather(xo_v, [rowvec + (kb + t)]) for t in range(4)]
            es = [plsc.load_gather(wx_v, [gxv + (kb + t)]) for t in range(4)]
            for t in range(4):
                plsc.store_scatter(xo_v, [rowvec + (kb + t)], xs[t] + es[t])
            xs = [plsc.load_gather(xo_v, [rowvec2 + (kb + t)]) for t in range(4)]
            es = [plsc.load_gather(wy_v, [gyv + (kb + t)]) for t in range(4)]
            for t in range(4):
                plsc.store_scatter(xo_v, [rowvec2 + (kb + t)], xs[t] + es[t])
    pltpu.sync_copy(xo_v, out.at[pl.ds(base, ROWS * DIM)])


def kernel(x, coords, Wx, Wy):
    out = _pe_kernel(coords[:, 1], coords[:, 2], x.reshape(-1),
                     Wx.reshape(-1), Wy.reshape(-1))
    return out.reshape(SEQ, DIM)


# speedup vs baseline: 1.3773x; 1.0029x over previous
import functools

import jax
import jax.numpy as jnp
from jax import lax
from jax.experimental import pallas as pl
from jax.experimental.pallas import tpu as pltpu
from jax.experimental.pallas import tpu_sc as plsc

SEQ = 16384
DIM = 64
HALF = 32
MAX_LEN = 512
NC = 2
NS = 16
L = 16
CHUNK = SEQ // NS
ROWS = SEQ // (NC * NS)
INT_MAX = 2147483647

_mesh = plsc.VectorSubcoreMesh(core_axis_name="c", subcore_axis_name="s")


@functools.partial(
    pl.kernel,
    out_type=jax.ShapeDtypeStruct((SEQ * DIM,), jnp.float32),
    mesh=_mesh,
    compiler_params=pltpu.CompilerParams(needs_layout_passes=False),
    scratch_types=[
        pltpu.VMEM((CHUNK,), jnp.int32),
        pltpu.VMEM((CHUNK,), jnp.int32),
        pltpu.VMEM((L,), jnp.int32),
        pltpu.VMEM((L,), jnp.int32),
        pltpu.VMEM_SHARED((NS * L,), jnp.int32),
        pltpu.VMEM_SHARED((NS * L,), jnp.int32),
        pltpu.VMEM((NS * L,), jnp.int32),
        pltpu.VMEM((NS * L,), jnp.int32),
        pltpu.VMEM((2 * L,), jnp.int32),
        pltpu.VMEM((2 * L,), jnp.int32),
        pltpu.VMEM((MAX_LEN * HALF,), jnp.float32),
        pltpu.VMEM((MAX_LEN * HALF,), jnp.float32),
        pltpu.VMEM((ROWS * DIM,), jnp.float32),
        pltpu.SemaphoreType.DMA,
    ],
)
def _pe_kernel(cx, cy, x_flat, wx, wy, out, cx_v, cy_v, stx, sty,
               minx_sh, miny_sh, mgx, mgy, redx, redy,
               wx_v, wy_v, xo_v, sem):
    c = lax.axis_index("c")
    s = lax.axis_index("s")
    base = (s * CHUNK + c * ROWS) * DIM
    cp_x = pltpu.async_copy(x_flat.at[pl.ds(base, ROWS * DIM)], xo_v, sem)
    cp_wx = pltpu.async_copy(wx, wx_v, sem)
    cp_wy = pltpu.async_copy(wy, wy_v, sem)

    pltpu.sync_copy(cx.at[pl.ds(s * CHUNK, CHUNK)], cx_v)
    pltpu.sync_copy(cy.at[pl.ds(s * CHUNK, CHUNK)], cy_v)

    def min_body(j, carry):
        mx, my = carry
        vx = cx_v[pl.ds(j * L, L)]
        vy = cy_v[pl.ds(j * L, L)]
        return jnp.minimum(mx, vx), jnp.minimum(my, vy)

    init = (jnp.full((L,), INT_MAX, jnp.int32), jnp.full((L,), INT_MAX, jnp.int32))
    with jax.named_scope("minphase"):
        mx, my = lax.fori_loop(0, CHUNK // L, min_body, init)

    stx[...] = mx
    sty[...] = my
    pltpu.sync_copy(stx, minx_sh.at[pl.ds(s * L, L)])
    pltpu.sync_copy(sty, miny_sh.at[pl.ds(s * L, L)])
    plsc.subcore_barrier()
    pltpu.sync_copy(minx_sh, mgx)
    pltpu.sync_copy(miny_sh, mgy)
    vx = mgx[pl.ds(0, L)]
    vy = mgy[pl.ds(0, L)]
    for j in range(1, NS):
        vx = jnp.minimum(vx, mgx[pl.ds(j * L, L)])
        vy = jnp.minimum(vy, mgy[pl.ds(j * L, L)])
    redx[pl.ds(0, L)] = vx
    redx[pl.ds(L, L)] = vx
    redy[pl.ds(0, L)] = vy
    redy[pl.ds(L, L)] = vy
    gmx = redx[pl.ds(0, L)]
    gmy = redy[pl.ds(0, L)]
    for k in range(1, L):
        gmx = jnp.minimum(gmx, redx[pl.ds(k, L)])
        gmy = jnp.minimum(gmy, redy[pl.ds(k, L)])

    with jax.named_scope("dma_wait"):
        cp_x.wait()
        cp_wx.wait()
        cp_wy.wait()

    off = c * ROWS
    shift4 = jnp.full((L,), 4, jnp.int32)
    shift5 = jnp.full((L,), 5, jnp.int32)
    lanes = lax.iota(jnp.int32, L)

    scope_add = jax.named_scope("addloop")
    scope_add.__enter__()

    @plsc.parallel_loop(0, ROWS // L, 1, unroll=2)
    def add_body(j):
        vx = cx_v[pl.ds(off + j * L, L)]
        vy = cy_v[pl.ds(off + j * L, L)]
        gxv = lax.shift_left(lax.shift_right_logical(vx - gmx, shift4), shift5)
        gyv = lax.shift_left(lax.shift_right_logical(vy - gmy, shift4), shift5)
        rowvec = (j * L + lanes) * DIM
        rowvec2 = rowvec + HALF
        for kb in range(0, HALF, 4):
            xs = [plsc.load_gather(xo_v, [rowvec + (kb + t)]) for t in range(4)]
            es = [plsc.load_gather(wx_v, [gxv + (kb + t)]) for t in range(4)]
            for t in range(4):
                plsc.store_scatter(xo_v, [rowvec + (kb + t)], xs[t] + es[t])
            xs = [plsc.load_gather(xo_v, [rowvec2 + (kb + t)]) for t in range(4)]
            es = [plsc.load_gather(wy_v, [gyv + (kb + t)]) for t in range(4)]
            for t in range(4):
                plsc.store_scatter(xo_v, [rowvec2 + (kb + t)], xs[t] + es[t])
    scope_add.__exit__(None, None, None)
    with jax.named_scope("writeback"):
        pltpu.sync_copy(xo_v, out.at[pl.ds(base, ROWS * DIM)])


def kernel(x, coords, Wx, Wy):
    out = _pe_kernel(coords[:, 1], coords[:, 2], x.reshape(-1),
                     Wx.reshape(-1), Wy.reshape(-1))
    return out.reshape(SEQ, DIM)


# speedup vs baseline: 1.7937x; 1.3024x over previous
import functools

import jax
import jax.numpy as jnp
from jax import lax
from jax.experimental import pallas as pl
from jax.experimental.pallas import tpu as pltpu
from jax.experimental.pallas import tpu_sc as plsc

SEQ = 16384
DIM = 64
HALF = 32
MAX_LEN = 512
NC = 2
NS = 16
L = 16
CHUNK = SEQ // NS
ROWS = SEQ // (NC * NS)
INT_MAX = 2147483647

_mesh = plsc.VectorSubcoreMesh(core_axis_name="c", subcore_axis_name="s")


@functools.partial(
    pl.kernel,
    out_type=jax.ShapeDtypeStruct((SEQ * DIM,), jnp.float32),
    mesh=_mesh,
    compiler_params=pltpu.CompilerParams(needs_layout_passes=False),
    scratch_types=[
        pltpu.VMEM((CHUNK * 3,), jnp.int32),
        pltpu.VMEM((L,), jnp.int32),
        pltpu.VMEM((L,), jnp.int32),
        pltpu.VMEM_SHARED((NS * L,), jnp.int32),
        pltpu.VMEM_SHARED((NS * L,), jnp.int32),
        pltpu.VMEM((NS * L,), jnp.int32),
        pltpu.VMEM((NS * L,), jnp.int32),
        pltpu.VMEM((2 * L,), jnp.int32),
        pltpu.VMEM((2 * L,), jnp.int32),
        pltpu.VMEM((MAX_LEN * HALF,), jnp.float32),
        pltpu.VMEM((MAX_LEN * HALF,), jnp.float32),
        pltpu.VMEM((ROWS * DIM,), jnp.float32),
        pltpu.SemaphoreType.DMA,
    ],
)
def _pe_kernel(coords, x_flat, wx, wy, out, co_v, stx, sty,
               minx_sh, miny_sh, mgx, mgy, redx, redy,
               wx_v, wy_v, xo_v, sem):
    c = lax.axis_index("c")
    s = lax.axis_index("s")
    base = (s * CHUNK + c * ROWS) * DIM
    HD = ROWS * DIM // 2
    cp_x0 = pltpu.async_copy(x_flat.at[pl.ds(base, HD)], xo_v.at[pl.ds(0, HD)], sem)
    cp_x1 = pltpu.async_copy(x_flat.at[pl.ds(base + HD, HD)],
                             xo_v.at[pl.ds(HD, HD)], sem)
    cp_wx = pltpu.async_copy(wx, wx_v, sem)
    cp_wy = pltpu.async_copy(wy, wy_v, sem)

    pltpu.sync_copy(coords.at[pl.ds(s * CHUNK * 3, CHUNK * 3)], co_v)

    lanes = lax.iota(jnp.int32, L)
    lanes3 = lanes * 3

    def min_body(j, carry):
        mx, my = carry
        p = j * (L * 3) + lanes3
        vx = plsc.load_gather(co_v, [p + 1])
        vy = plsc.load_gather(co_v, [p + 2])
        return jnp.minimum(mx, vx), jnp.minimum(my, vy)

    init = (jnp.full((L,), INT_MAX, jnp.int32), jnp.full((L,), INT_MAX, jnp.int32))
    with jax.named_scope("minphase"):
        mx, my = lax.fori_loop(0, CHUNK // L, min_body, init)

    stx[...] = mx
    sty[...] = my
    pltpu.sync_copy(stx, minx_sh.at[pl.ds(s * L, L)])
    pltpu.sync_copy(sty, miny_sh.at[pl.ds(s * L, L)])
    plsc.subcore_barrier()
    pltpu.sync_copy(minx_sh, mgx)
    pltpu.sync_copy(miny_sh, mgy)
    vx = mgx[pl.ds(0, L)]
    vy = mgy[pl.ds(0, L)]
    for j in range(1, NS):
        vx = jnp.minimum(vx, mgx[pl.ds(j * L, L)])
        vy = jnp.minimum(vy, mgy[pl.ds(j * L, L)])
    redx[pl.ds(0, L)] = vx
    redx[pl.ds(L, L)] = vx
    redy[pl.ds(0, L)] = vy
    redy[pl.ds(L, L)] = vy
    gmx = redx[pl.ds(0, L)]
    gmy = redy[pl.ds(0, L)]
    for k in range(1, L):
        gmx = jnp.minimum(gmx, redx[pl.ds(k, L)])
        gmy = jnp.minimum(gmy, redy[pl.ds(k, L)])

    off = c * ROWS
    shift4 = jnp.full((L,), 4, jnp.int32)
    shift5 = jnp.full((L,), 5, jnp.int32)

    def add_group(j):
        p = (off + j * L) * 3 + lanes3
        vx = plsc.load_gather(co_v, [p + 1])
        vy = plsc.load_gather(co_v, [p + 2])
        gxv = lax.shift_left(lax.shift_right_logical(vx - gmx, shift4), shift5)
        gyv = lax.shift_left(lax.shift_right_logical(vy - gmy, shift4), shift5)
        o = j * (L * DIM)
        for t in range(L):
            ox = gxv[t]
            oy = gyv[t]
            r = o + t * DIM
            xo_v[pl.ds(r, L)] = xo_v[pl.ds(r, L)] + wx_v[pl.ds(ox, L)]
            xo_v[pl.ds(r + 16, L)] = (
                xo_v[pl.ds(r + 16, L)] + wx_v[pl.ds(ox + 16, L)])
            xo_v[pl.ds(r + 32, L)] = xo_v[pl.ds(r + 32, L)] + wy_v[pl.ds(oy, L)]
            xo_v[pl.ds(r + 48, L)] = (
                xo_v[pl.ds(r + 48, L)] + wy_v[pl.ds(oy + 16, L)])

    NG = ROWS // L
    with jax.named_scope("dma_wait0"):
        cp_wx.wait()
        cp_wy.wait()
        cp_x0.wait()

    scope_add = jax.named_scope("addloop0")
    scope_add.__enter__()

    @plsc.parallel_loop(0, NG // 2, 1, unroll=1)
    def add_body0(j):
        add_group(j)

    scope_add.__exit__(None, None, None)
    with jax.named_scope("dma_wait1"):
        cp_x1.wait()
    scope_add = jax.named_scope("addloop1")
    scope_add.__enter__()

    @plsc.parallel_loop(NG // 2, NG, 1, unroll=1)
    def add_body1(j):
        add_group(j)

    scope_add.__exit__(None, None, None)
    with jax.named_scope("writeback"):
        pltpu.sync_copy(xo_v, out.at[pl.ds(base, ROWS * DIM)])


def kernel(x, coords, Wx, Wy):
    out = _pe_kernel(coords.reshape(-1), x.reshape(-1),
                     Wx.reshape(-1), Wy.reshape(-1))
    return out.reshape(SEQ, DIM)
